# Initial kernel scaffold; baseline (speedup 1.0000x reference)
import jax
import jax.numpy as jnp
from jax.experimental import pallas as pl


def kernel(inputs, weight, bias, row1, col1, val1, row2, col2, val2):
    raise NotImplementedError("write your pallas kernel here")


# speedup vs baseline: 1.0122x; 1.0122x over previous
import jax
import jax.numpy as jnp
from jax.experimental import pallas as pl
from jax.experimental.pallas import tpu as pltpu

N = 10000
P = 128
Q = 128
B = 4
MM = 5
TN = 1000


def _dense_body(x0_ref, xs_ref, w_ref, b_ref, o_ref):
    acc = jnp.dot(x0_ref[0], w_ref[0], preferred_element_type=jnp.float32)
    for m in range(4):
        acc += jnp.dot(xs_ref[m, 0], w_ref[m + 1],
                       preferred_element_type=jnp.float32)
    o_ref[0] = acc + b_ref[...]


def _dense_stage(x0, xs, wmod, bias):
    grid = (B, N // TN)
    return pl.pallas_call(
        _dense_body,
        grid=grid,
        in_specs=[
            pl.BlockSpec((1, TN, P), lambda b, i: (b, i, 0)),
            pl.BlockSpec((4, 1, TN, P), lambda b, i: (0, b, i, 0)),
            pl.BlockSpec((MM, P, Q), lambda b, i: (0, 0, 0)),
            pl.BlockSpec((1, Q), lambda b, i: (0, 0)),
        ],
        out_specs=pl.BlockSpec((1, TN, Q), lambda b, i: (b, i, 0)),
        out_shape=jax.ShapeDtypeStruct((B, N, Q), jnp.float32),
    )(x0, xs, wmod, bias)


def _spmm_flat(r, c, v, x):
    return jax.ops.segment_sum(v[:, None] * x[c], r, num_segments=N)


def kernel(inputs, weight, bias, row1, col1, val1, row2, col2, val2):
    x0 = inputs.reshape(B, N, P)
    x0f = inputs.reshape(B, N * P).reshape(B, N, P).transpose(1, 2, 0).reshape(N, P * B)
    x11f = _spmm_flat(row1, col1, val1, x0f)
    y1f = _spmm_flat(row1, col1, val1, x11f)
    x12f = _spmm_flat(row2, col2, val2, x0f)
    y2f = _spmm_flat(row2, col2, val2, x12f)
    xs = jnp.stack([
        x.reshape(N, P, B).transpose(2, 0, 1) for x in (x11f, y1f, x12f, y2f)
    ])
    w5 = weight.reshape(P, MM, Q).transpose(1, 0, 2)
    wmod = jnp.stack([w5[0] - w5[2] - w5[4], w5[1], 2.0 * w5[2],
                      w5[3], 2.0 * w5[4]])
    out = _dense_stage(x0, xs, wmod, bias.reshape(1, Q))
    return out.reshape(B, N * Q)


# speedup vs baseline: 2.5246x; 2.4942x over previous
import jax
import jax.numpy as jnp
from jax import lax
from jax.experimental import pallas as pl
from jax.experimental.pallas import tpu as pltpu
from jax.experimental.pallas import tpu_sc as plsc

N = 10000
P = 128
Q = 128
B = 4
MM = 5
E = 320000
NSUB = 16
NB = 159
EPT_PAD = NB * 128
NACC = 10008
WPT = 624
XR = 4 * B * N
TN = 1000


def _scale_buf(gb, vals, s):
    def _sb(j16, _):
        vvec = vals[s, pl.ds(j16 * 16, 16)]
        for l in range(16):
            vv = jnp.full((16,), vvec[l], jnp.float32)
            j = j16 * 16 + l
            for t in range(8):
                sl = pl.ds(t * 16, 16)
                gb[j, sl] = gb[j, sl] * vv
        return 0
    lax.fori_loop(0, 8, _sb, 0)


def _sc_body(x0f, ecol, erow, evalv, outf,
             acc, cols, rows, vals, gb0, gb1, gb2,
             gsem0, gsem1, gsem2, ssem0, ssem1, ssem2,
             csem0, csem1, csem2, vsem0, vsem1, vsem2,
             rsem0, rsem1, rsem2):
    cid = lax.axis_index("c")
    sid = lax.axis_index("s")
    gb = (gb0, gb1, gb2)
    gsem = (gsem0, gsem1, gsem2)
    ssem = (ssem0, ssem1, ssem2)
    csem = (csem0, csem1, csem2)
    vsem = (vsem0, vsem1, vsem2)
    rsem = (rsem0, rsem1, rsem2)

    x0 = pl.multiple_of(2 * cid * N, 8)

    @pl.when(sid < NSUB - 1)
    def _cp_small():
        st = pl.multiple_of(x0 + sid * 1248, 8)
        pltpu.sync_copy(x0f.at[pl.ds(st, 1248)],
                        outf.at[pl.ds(XR + st, 1248)])

    @pl.when(sid == NSUB - 1)
    def _cp_last():
        st = pl.multiple_of(x0 + 18720, 8)
        pltpu.sync_copy(x0f.at[pl.ds(st, 1280)],
                        outf.at[pl.ds(XR + st, 1280)])

    plsc.subcore_barrier()

    def _pass(q, _):
        es = q // 4
        rnd = (q // 2) % 2
        bb = q % 2
        b = 2 * cid + bb
        srcreg = jnp.where(rnd == 0, 4, 2 * es)
        base = (srcreg * B + b) * N
        dstbase = ((2 * es + rnd) * B + b) * N

        def _zb(j, _):
            for t in range(8):
                gb0[j, pl.ds(t * 16, 16)] = jnp.zeros((16,), jnp.float32)
            return 0
        lax.fori_loop(0, 128, _zb, 0)
        z0 = pl.multiple_of(sid * WPT, 8)

        def _zc(k, _):
            zo = pl.multiple_of(z0 + k * 128, 8)
            pltpu.sync_copy(gb0, acc.at[pl.ds(zo, 128)])
            return 0
        lax.fori_loop(0, 4, _zc, 0)

        @pl.when(sid < NSUB - 1)
        def _z_small():
            zo = pl.multiple_of(z0 + 512, 8)
            pltpu.sync_copy(gb0.at[pl.ds(0, 112)], acc.at[pl.ds(zo, 112)])

        @pl.when(sid == NSUB - 1)
        def _z_last():
            zo = pl.multiple_of(z0 + 512, 8)
            pltpu.sync_copy(gb0, acc.at[pl.ds(zo, 128)])
            zo2 = pl.multiple_of(z0 + 640, 8)
            pltpu.sync_copy(gb0.at[pl.ds(0, 8)], acc.at[pl.ds(zo2, 8)])

        plsc.subcore_barrier()

        def _bias(sj):
            for t in range(8):
                sl = pl.ds(t * 16, 16)
                cols[sj, sl] = cols[sj, sl] + base

        for s in range(3):
            pltpu.async_copy(ecol.at[es, sid, s], cols.at[s], csem[s])
            pltpu.async_copy(evalv.at[es, sid, s], vals.at[s], vsem[s])
            pltpu.async_copy(erow.at[es, sid, s], rows.at[s], rsem[s])
        for s in range(2):
            pltpu.make_async_copy(ecol.at[0, sid, 0], cols.at[s],
                                  csem[s]).wait()
            _bias(s)
            pltpu.async_copy(outf.at[cols.at[s]], gb[s], gsem[s])

        def _grp(g, _):
            for s in range(3):
                i = 3 * g + s
                sj = (s + 2) % 3
                pltpu.make_async_copy(outf.at[cols.at[s]], gb[s],
                                      gsem[s]).wait()
                pltpu.make_async_copy(ecol.at[0, sid, 0], vals.at[s],
                                      vsem[s]).wait()
                _scale_buf(gb[s], vals, s)
                pltpu.make_async_copy(ecol.at[0, sid, 0], rows.at[s],
                                      rsem[s]).wait()
                pltpu.async_copy(gb[s], acc.at[rows.at[s]], ssem[s],
                                 add=True)

                @pl.when(i + 3 <= NB - 1)
                def _ld():
                    pltpu.async_copy(ecol.at[es, sid, i + 3], cols.at[s],
                                     csem[s])
                    pltpu.async_copy(evalv.at[es, sid, i + 3], vals.at[s],
                                     vsem[s])

                j = i + 2

                @pl.when(j <= NB - 1)
                def _prep():
                    @pl.when(j >= 3)
                    def _free():
                        pltpu.make_async_copy(gb[sj], acc.at[rows.at[sj]],
                                              ssem[sj]).wait()
                        pltpu.async_copy(erow.at[es, sid, j], rows.at[sj],
                                         rsem[sj])
                    pltpu.make_async_copy(ecol.at[0, sid, 0], cols.at[sj],
                                          csem[sj]).wait()
                    _bias(sj)
                    pltpu.async_copy(outf.at[cols.at[sj]], gb[sj], gsem[sj])
            return 0
        lax.fori_loop(0, NB // 3, _grp, 0)

        for s in range(3):
            pltpu.make_async_copy(gb[s], acc.at[rows.at[s]], ssem[s]).wait()
        plsc.subcore_barrier()

        w0 = pl.multiple_of(sid * WPT, 8)
        db = pl.multiple_of(dstbase + sid * WPT, 8)

        def _wc(k, _):
            wo = pl.multiple_of(w0 + k * 128, 8)
            do = pl.multiple_of(db + k * 128, 8)
            pltpu.sync_copy(acc.at[pl.ds(wo, 128)], gb0)
            pltpu.sync_copy(gb0, outf.at[pl.ds(do, 128)])
            return 0
        lax.fori_loop(0, 4, _wc, 0)

        @pl.when(sid < NSUB - 1)
        def _wb_small():
            wo = pl.multiple_of(w0 + 512, 8)
            do = pl.multiple_of(db + 512, 8)
            pltpu.sync_copy(acc.at[pl.ds(wo, WPT - 512)],
                            gb0.at[pl.ds(0, WPT - 512)])
            pltpu.sync_copy(gb0.at[pl.ds(0, WPT - 512)],
                            outf.at[pl.ds(do, WPT - 512)])

        @pl.when(sid == NSUB - 1)
        def _wb_last():
            wo = pl.multiple_of(w0 + 512, 8)
            do = pl.multiple_of(db + 512, 8)
            pltpu.sync_copy(acc.at[pl.ds(wo, 128)], gb0)
            pltpu.sync_copy(gb0, outf.at[pl.ds(do, 128)])

        plsc.subcore_barrier()
        return 0

    lax.fori_loop(0, 8, _pass, 0)


def _sc_stage(x0f, ecol, erow, evalv):
    mesh = plsc.VectorSubcoreMesh(core_axis_name="c", subcore_axis_name="s")
    fn = pl.kernel(
        _sc_body,
        out_type=jax.ShapeDtypeStruct((5 * B * N, P), jnp.float32),
        mesh=mesh,
        scratch_types=[
            pltpu.VMEM_SHARED((NACC, P), jnp.float32),
            pltpu.VMEM((3, 128), jnp.int32),
            pltpu.VMEM((3, 128), jnp.int32),
            pltpu.VMEM((3, 128), jnp.float32),
            pltpu.VMEM((128, 128), jnp.float32),
            pltpu.VMEM((128, 128), jnp.float32),
            pltpu.VMEM((128, 128), jnp.float32),
        ] + [pltpu.SemaphoreType.DMA] * 15,
    )
    return fn(x0f, ecol, erow, evalv)


def _pad_edges(col, row, val):
    pad = EPT_PAD * NSUB - E
    col = jnp.concatenate([col.astype(jnp.int32), jnp.zeros((pad,), jnp.int32)])
    row = jnp.concatenate([row.astype(jnp.int32),
                           jnp.full((pad,), N, jnp.int32)])
    val = jnp.concatenate([val, jnp.zeros((pad,), jnp.float32)])
    return (col.reshape(NSUB, NB, 128), row.reshape(NSUB, NB, 128),
            val.reshape(NSUB, NB, 128))


def _dense_body(x0_ref, xs_ref, w_ref, b_ref, o_ref):
    acc = jnp.dot(x0_ref[0], w_ref[0], preferred_element_type=jnp.float32)
    for m in range(4):
        acc += jnp.dot(xs_ref[m, 0], w_ref[m + 1],
                       preferred_element_type=jnp.float32)
    o_ref[0] = acc + b_ref[...]


def _dense_stage(x0, xs5, wmod, bias):
    grid = (B, N // TN)
    return pl.pallas_call(
        _dense_body,
        grid=grid,
        in_specs=[
            pl.BlockSpec((1, TN, P), lambda b, i: (b, i, 0)),
            pl.BlockSpec((4, 1, TN, P), lambda b, i: (0, b, i, 0)),
            pl.BlockSpec((MM, P, Q), lambda b, i: (0, 0, 0)),
            pl.BlockSpec((1, Q), lambda b, i: (0, 0)),
        ],
        out_specs=pl.BlockSpec((1, TN, Q), lambda b, i: (b, i, 0)),
        out_shape=jax.ShapeDtypeStruct((B, N, Q), jnp.float32),
    )(x0, xs5, wmod, bias)


def kernel(inputs, weight, bias, row1, col1, val1, row2, col2, val2):
    x0 = inputs.reshape(B, N, P)
    c1, r1, v1 = _pad_edges(col1, row1, val1)
    c2, r2, v2 = _pad_edges(col2, row2, val2)
    ecol = jnp.stack([c1, c2])
    erow = jnp.stack([r1, r2])
    evalv = jnp.stack([v1, v2])
    xsf = _sc_stage(x0.reshape(B * N, P), ecol, erow, evalv)
    xs5 = xsf.reshape(5, B, N, P)
    w5 = weight.reshape(P, MM, Q).transpose(1, 0, 2)
    wmod = jnp.stack([w5[0] - w5[2] - w5[4], w5[1], 2.0 * w5[2],
                      w5[3], 2.0 * w5[4]])
    out = _dense_stage(x0, xs5, wmod, bias.reshape(1, Q))
    return out.reshape(B, N * Q)
